# Initial kernel scaffold; baseline (speedup 1.0000x reference)
import jax
import jax.numpy as jnp
from jax.experimental import pallas as pl


def kernel(input, vote_mapping):
    raise NotImplementedError("write your pallas kernel here")


# speedup vs baseline: 52.8751x; 52.8751x over previous
import functools

import jax
import jax.numpy as jnp
from jax import lax
from jax.experimental import pallas as pl
from jax.experimental.pallas import tpu as pltpu
from jax.experimental.pallas import tpu_sc as plsc

B, C = 2, 64
HT_BINS = 184 * 60
IM_BINS = 128 * 128
N_VOTES = 262144
P = B * C

NC, NS, L = 2, 16, 16
NW = NC * NS
CPW = P // NW

CHUNK = 2048
NCHUNK = N_VOTES // CHUNK
STEPS = CHUNK // L


def _ht2im_body(tbl_hbm, ht_hbm, im_hbm, w_hbm, out_hbm,
                table_v, accum_v,
                ht0, im0, w0, ht1, im1, w1, sem0, sem1):
    wid = lax.axis_index("s") * NC + lax.axis_index("c")

    pltpu.sync_copy(tbl_hbm.at[pl.ds(wid * CPW * HT_BINS, CPW * HT_BINS)],
                    table_v)

    zv = jnp.zeros((L,), jnp.float32)
    def zstep(i, _):
        accum_v[pl.ds(i * L, L)] = zv
        return 0
    lax.fori_loop(0, (CPW * IM_BINS) // L, zstep, 0)

    def start(g, bufs, sem):
        htb, imb, wb = bufs
        off = g * CHUNK
        pltpu.async_copy(ht_hbm.at[pl.ds(off, CHUNK)], htb, sem)
        pltpu.async_copy(im_hbm.at[pl.ds(off, CHUNK)], imb, sem)
        pltpu.async_copy(w_hbm.at[pl.ds(off, CHUNK)], wb, sem)

    def wait(bufs, sem):
        htb, imb, wb = bufs
        pltpu.make_async_copy(ht_hbm.at[pl.ds(0, CHUNK)], htb, sem).wait()
        pltpu.make_async_copy(im_hbm.at[pl.ds(0, CHUNK)], imb, sem).wait()
        pltpu.make_async_copy(w_hbm.at[pl.ds(0, CHUNK)], wb, sem).wait()

    def compute(bufs):
        htb, imb, wb = bufs
        def step(i, _):
            base = i * L
            ht = htb[pl.ds(base, L)]
            im = imb[pl.ds(base, L)]
            w = wb[pl.ds(base, L)]
            for c in range(CPW):
                g = plsc.load_gather(table_v, [ht + (c * HT_BINS)])
                plsc.addupdate_scatter(accum_v, [im + (c * IM_BINS)], g * w)
            return 0
        lax.fori_loop(0, STEPS, step, 0)

    bufs0 = (ht0, im0, w0)
    bufs1 = (ht1, im1, w1)

    start(0, bufs0, sem0)

    def outer(gg, _):
        g0 = gg * 2
        start(g0 + 1, bufs1, sem1)
        wait(bufs0, sem0)
        compute(bufs0)

        @pl.when(gg + 1 < NCHUNK // 2)
        def _():
            start(g0 + 2, bufs0, sem0)

        wait(bufs1, sem1)
        compute(bufs1)
        return 0

    lax.fori_loop(0, NCHUNK // 2, outer, 0)

    pltpu.sync_copy(accum_v, out_hbm.at[pl.ds(wid * CPW * IM_BINS,
                                              CPW * IM_BINS)])


@jax.jit
def _ht2im(tbl, ht, im, w):
    mesh = plsc.VectorSubcoreMesh(
        core_axis_name="c", subcore_axis_name="s",
        num_cores=NC, num_subcores=NS)
    run = pl.kernel(
        _ht2im_body,
        out_type=jax.ShapeDtypeStruct((P * IM_BINS,), jnp.float32),
        mesh=mesh,
        compiler_params=pltpu.CompilerParams(needs_layout_passes=False),
        scratch_types=[
            pltpu.VMEM((CPW * HT_BINS,), jnp.float32),
            pltpu.VMEM((CPW * IM_BINS,), jnp.float32),
            pltpu.VMEM((CHUNK,), jnp.int32),
            pltpu.VMEM((CHUNK,), jnp.int32),
            pltpu.VMEM((CHUNK,), jnp.float32),
            pltpu.VMEM((CHUNK,), jnp.int32),
            pltpu.VMEM((CHUNK,), jnp.int32),
            pltpu.VMEM((CHUNK,), jnp.float32),
            pltpu.SemaphoreType.DMA,
            pltpu.SemaphoreType.DMA,
        ],
    )
    return run(tbl, ht, im, w)


def kernel(input, vote_mapping):
    b, c, hh, hw = input.shape
    tbl = input.reshape(b * c * hh * hw)
    ht = vote_mapping[:, 0].astype(jnp.int32)
    im = vote_mapping[:, 1].astype(jnp.int32)
    w = vote_mapping[:, 2]
    out = _ht2im(tbl, ht, im, w)
    return out.reshape(b, c, 128, 128)


# speedup vs baseline: 124.0044x; 2.3452x over previous
import functools

import jax
import jax.numpy as jnp
from jax import lax
from jax.experimental import pallas as pl
from jax.experimental.pallas import tpu as pltpu
from jax.experimental.pallas import tpu_sc as plsc

B, C = 2, 64
HT_BINS = 184 * 60
IM_BINS = 128 * 128
N_VOTES = 262144
P = B * C

NC, NS, L = 2, 16, 16
NW = NC * NS
CPW = P // NW

CHUNK = 2048
NCHUNK = N_VOTES // CHUNK
STEPS = CHUNK // L


def _ht2im_body(tbl_hbm, ht_hbm, im_hbm, w_hbm, out_hbm,
                table_v, accum_v,
                ht0, im0, w0, ht1, im1, w1, sem0, sem1):
    wid = lax.axis_index("s") * NC + lax.axis_index("c")

    pltpu.sync_copy(tbl_hbm.at[pl.ds(wid * CPW * HT_BINS, CPW * HT_BINS)],
                    table_v)

    zv = jnp.zeros((L,), jnp.float32)

    @plsc.parallel_loop(0, CPW * IM_BINS, step=L, unroll=8)
    def _zero(i):
        accum_v[pl.ds(i, L)] = zv

    def start(g, bufs, sem):
        htb, imb, wb = bufs
        off = g * CHUNK
        pltpu.async_copy(ht_hbm.at[pl.ds(off, CHUNK)], htb, sem)
        pltpu.async_copy(im_hbm.at[pl.ds(off, CHUNK)], imb, sem)
        pltpu.async_copy(w_hbm.at[pl.ds(off, CHUNK)], wb, sem)

    def wait(bufs, sem):
        htb, imb, wb = bufs
        pltpu.make_async_copy(ht_hbm.at[pl.ds(0, CHUNK)], htb, sem).wait()
        pltpu.make_async_copy(im_hbm.at[pl.ds(0, CHUNK)], imb, sem).wait()
        pltpu.make_async_copy(w_hbm.at[pl.ds(0, CHUNK)], wb, sem).wait()

    def compute(bufs):
        htb, imb, wb = bufs

        @plsc.parallel_loop(0, CHUNK, step=L, unroll=4)
        def _steps(base):
            ht = htb[pl.ds(base, L)]
            im = imb[pl.ds(base, L)]
            w = wb[pl.ds(base, L)]
            for c in range(CPW):
                g = plsc.load_gather(table_v, [ht + (c * HT_BINS)])
                plsc.addupdate_scatter(accum_v, [im + (c * IM_BINS)], g * w)

    bufs0 = (ht0, im0, w0)
    bufs1 = (ht1, im1, w1)

    start(0, bufs0, sem0)

    def outer(gg, _):
        g0 = gg * 2
        start(g0 + 1, bufs1, sem1)
        wait(bufs0, sem0)
        compute(bufs0)

        @pl.when(gg + 1 < NCHUNK // 2)
        def _():
            start(g0 + 2, bufs0, sem0)

        wait(bufs1, sem1)
        compute(bufs1)
        return 0

    lax.fori_loop(0, NCHUNK // 2, outer, 0)

    pltpu.sync_copy(accum_v, out_hbm.at[pl.ds(wid * CPW * IM_BINS,
                                              CPW * IM_BINS)])


@jax.jit
def _ht2im(tbl, ht, im, w):
    mesh = plsc.VectorSubcoreMesh(
        core_axis_name="c", subcore_axis_name="s",
        num_cores=NC, num_subcores=NS)
    run = pl.kernel(
        _ht2im_body,
        out_type=jax.ShapeDtypeStruct((P * IM_BINS,), jnp.float32),
        mesh=mesh,
        compiler_params=pltpu.CompilerParams(needs_layout_passes=False),
        scratch_types=[
            pltpu.VMEM((CPW * HT_BINS,), jnp.float32),
            pltpu.VMEM((CPW * IM_BINS,), jnp.float32),
            pltpu.VMEM((CHUNK,), jnp.int32),
            pltpu.VMEM((CHUNK,), jnp.int32),
            pltpu.VMEM((CHUNK,), jnp.float32),
            pltpu.VMEM((CHUNK,), jnp.int32),
            pltpu.VMEM((CHUNK,), jnp.int32),
            pltpu.VMEM((CHUNK,), jnp.float32),
            pltpu.SemaphoreType.DMA,
            pltpu.SemaphoreType.DMA,
        ],
    )
    return run(tbl, ht, im, w)


def kernel(input, vote_mapping):
    b, c, hh, hw = input.shape
    tbl = input.reshape(b * c * hh * hw)
    ht = vote_mapping[:, 0].astype(jnp.int32)
    im = vote_mapping[:, 1].astype(jnp.int32)
    w = vote_mapping[:, 2]
    out = _ht2im(tbl, ht, im, w)
    return out.reshape(b, c, 128, 128)
